# baseline (speedup 1.0000x reference)
import functools

import jax
import jax.numpy as jnp
from jax import lax
from jax.experimental import pallas as pl
from jax.experimental.pallas import tpu as pltpu
from jax.experimental.pallas import tpu_sc as plsc

B = 16384
HL = 50
D = 32
H = 64
O = 2

NC = 2
NS = 16
NW = NC * NS

IDXW = 128
IDX_ROWS = B * HL // IDXW
RPW = 4
ROWS_PER_W = IDX_ROWS // NW
CHUNKS = ROWS_PER_W // RPW
PAIRS = CHUNKS // 2
PERIOD = 25
B_PER_W = B // NW
B_PER_SC = B // NC
LANES = 16


def _sc_pool(emb_table, src_idx, pattern, zeros):
    mesh = plsc.VectorSubcoreMesh(core_axis_name="c", subcore_axis_name="s")

    @functools.partial(
        pl.kernel,
        out_type=jax.ShapeDtypeStruct((B, D), jnp.float32),
        mesh=mesh,
        compiler_params=pltpu.CompilerParams(use_tc_tiling_on_sc=False),
        scratch_types=[
            pltpu.VMEM((RPW, IDXW), jnp.int32),
            pltpu.VMEM((RPW, IDXW), jnp.int32),
            pltpu.VMEM((RPW, IDXW), jnp.int32),
            pltpu.VMEM((RPW, IDXW), jnp.int32),
            pltpu.VMEM((RPW, IDXW, D), jnp.float32),
            pltpu.VMEM((RPW, IDXW, D), jnp.float32),
            pltpu.VMEM((PERIOD, IDXW), jnp.int32),
            pltpu.VMEM((B_PER_W, D), jnp.float32),
            pltpu.VMEM_SHARED((B_PER_SC, D), jnp.float32),
            pltpu.SemaphoreType.DMA,
            pltpu.SemaphoreType.DMA,
            pltpu.SemaphoreType.DMA,
            pltpu.SemaphoreType.DMA,
        ],
    )
    def k(table_hbm, src_hbm, pat_hbm, zero_hbm, out_hbm,
          idx_a, idx_b, dst_a, dst_b, rows_a, rows_b, pat_v, stage, acc,
          sem_ga, sem_gb, sem_sa, sem_sb):
        c = lax.axis_index("c")
        s = lax.axis_index("s")
        wid = c * NS + s

        pltpu.sync_copy(pat_hbm, pat_v)
        pltpu.sync_copy(zero_hbm, acc.at[pl.ds(s * B_PER_W, B_PER_W)])
        plsc.subcore_barrier()

        def fire_gathers(chunk, idx, dst, rows, sem):
            pltpu.sync_copy(
                src_hbm.at[pl.ds(wid * ROWS_PER_W + chunk * RPW, RPW)], idx)
            for j in range(RPW):
                m = chunk * RPW + j
                off = s * B_PER_W + (m // PERIOD) * 64
                pm = m % PERIOD
                for g in range(IDXW // LANES):
                    dst[j, pl.ds(g * LANES, LANES)] = (
                        pat_v[pm, pl.ds(g * LANES, LANES)] + off)
                pltpu.async_copy(table_hbm.at[idx.at[j]], rows.at[j], sem)

        def drain_gathers(idx, rows, sem):
            for j in range(RPW):
                pltpu.make_async_copy(table_hbm.at[idx.at[j]], rows.at[j],
                                      sem).wait()

        def fire_scatters(rows, dst, sem):
            for j in range(RPW):
                pltpu.async_copy(rows.at[j], acc.at[dst.at[j]], sem, add=True)

        def drain_scatters(rows, dst, sem):
            for j in range(RPW):
                pltpu.make_async_copy(rows.at[j], acc.at[dst.at[j]],
                                      sem).wait()

        fire_gathers(0, idx_a, dst_a, rows_a, sem_ga)

        def pair(i, _):
            drain_gathers(idx_a, rows_a, sem_ga)
            fire_scatters(rows_a, dst_a, sem_sa)

            @pl.when(i > 0)
            def _():
                drain_scatters(rows_b, dst_b, sem_sb)

            fire_gathers(2 * i + 1, idx_b, dst_b, rows_b, sem_gb)
            drain_gathers(idx_b, rows_b, sem_gb)
            fire_scatters(rows_b, dst_b, sem_sb)
            drain_scatters(rows_a, dst_a, sem_sa)

            @pl.when(i < PAIRS - 1)
            def _():
                fire_gathers(2 * i + 2, idx_a, dst_a, rows_a, sem_ga)

            return 0

        lax.fori_loop(0, PAIRS, pair, 0)
        drain_scatters(rows_b, dst_b, sem_sb)

        plsc.subcore_barrier()
        pltpu.sync_copy(acc.at[pl.ds(s * B_PER_W, B_PER_W)], stage)
        pltpu.sync_copy(stage, out_hbm.at[pl.ds(wid * B_PER_W, B_PER_W)])

    return k(emb_table, src_idx, pattern, zeros)


def _mlp_body(p_ref, w1_ref, b1_ref, w2_ref, b2_ref, o_ref):
    p = p_ref[...] * (1.0 / HL)
    h = jnp.dot(p, w1_ref[...], preferred_element_type=jnp.float32)
    h = jnp.maximum(h + b1_ref[...], 0.0)
    o = jnp.dot(h, w2_ref[...], preferred_element_type=jnp.float32)
    o_ref[...] = o + b2_ref[...]


def _tc_mlp(pooled, W1, b1, W2, b2):
    GB = 2048
    grid = (B // GB,)
    return pl.pallas_call(
        _mlp_body,
        grid=grid,
        in_specs=[
            pl.BlockSpec((GB, D), lambda i: (i, 0)),
            pl.BlockSpec((D, H), lambda i: (0, 0)),
            pl.BlockSpec((1, H), lambda i: (0, 0)),
            pl.BlockSpec((H, O), lambda i: (0, 0)),
            pl.BlockSpec((1, O), lambda i: (0, 0)),
        ],
        out_specs=pl.BlockSpec((GB, O), lambda i: (i, 0)),
        out_shape=jax.ShapeDtypeStruct((B, O), jnp.float32),
    )(pooled, W1, b1, W2, b2)


def kernel(text, emb_table, W1, b1, W2, b2):
    src_idx = text.astype(jnp.int32).reshape(IDX_ROWS, IDXW)
    pattern = (jnp.arange(PERIOD * IDXW, dtype=jnp.int32) // HL).reshape(
        PERIOD, IDXW)
    zeros = jnp.zeros((B_PER_W, D), jnp.float32)
    pooled = _sc_pool(emb_table, src_idx, pattern, zeros)
    return _tc_mlp(pooled, W1.astype(jnp.float32), b1.reshape(1, H),
                   W2.astype(jnp.float32), b2.reshape(1, O))


# speedup vs baseline: 2.7866x; 2.7866x over previous
import functools

import jax
import jax.numpy as jnp
from jax import lax
from jax.experimental import pallas as pl
from jax.experimental.pallas import tpu as pltpu
from jax.experimental.pallas import tpu_sc as plsc

B = 16384
HL = 50
D = 32
H = 64
O = 2

NC = 2
NS = 16
NW = NC * NS

IDXW = 128
IDX_ROWS = B * HL // IDXW
RPW = 10
ROWS_PER_W = IDX_ROWS // NW
CHUNKS = ROWS_PER_W // RPW
PAIRS = CHUNKS // 2
PERIOD = 25
B_PER_W = B // NW
B_PER_SC = B // NC
LANES = 16


def _sc_pool(emb_table, src_idx, pattern, zeros):
    mesh = plsc.VectorSubcoreMesh(core_axis_name="c", subcore_axis_name="s")

    @functools.partial(
        pl.kernel,
        out_type=jax.ShapeDtypeStruct((B, D), jnp.float32),
        mesh=mesh,
        compiler_params=pltpu.CompilerParams(use_tc_tiling_on_sc=False),
        scratch_types=[
            pltpu.VMEM((RPW, IDXW), jnp.int32),
            pltpu.VMEM((RPW, IDXW), jnp.int32),
            pltpu.VMEM((RPW, IDXW), jnp.int32),
            pltpu.VMEM((RPW, IDXW), jnp.int32),
            pltpu.VMEM((RPW, IDXW, D), jnp.float32),
            pltpu.VMEM((RPW, IDXW, D), jnp.float32),
            pltpu.VMEM((PERIOD, IDXW), jnp.int32),
            pltpu.VMEM((B_PER_W, D), jnp.float32),
            pltpu.VMEM_SHARED((B_PER_SC, D), jnp.float32),
            pltpu.SemaphoreType.DMA,
            pltpu.SemaphoreType.DMA,
            pltpu.SemaphoreType.DMA,
            pltpu.SemaphoreType.DMA,
            pltpu.SemaphoreType.DMA,
            pltpu.SemaphoreType.DMA,
        ],
    )
    def k(table_hbm, src_hbm, pat_hbm, zero_hbm, out_hbm,
          idx_a, idx_b, dst_a, dst_b, rows_a, rows_b, pat_v, stage, acc,
          sem_ga, sem_gb, sem_sa, sem_sb, sem_ia, sem_ib):
        c = lax.axis_index("c")
        s = lax.axis_index("s")
        wid = c * NS + s

        pltpu.sync_copy(pat_hbm, pat_v)
        pltpu.sync_copy(zero_hbm, acc.at[pl.ds(s * B_PER_W, B_PER_W)])
        plsc.subcore_barrier()

        def fire_idx(chunk, idx, sem):
            pltpu.async_copy(
                src_hbm.at[pl.ds(wid * ROWS_PER_W + chunk * RPW, RPW)], idx,
                sem)

        def wait_idx(idx, sem):
            pltpu.make_async_copy(src_hbm.at[pl.ds(0, RPW)], idx, sem).wait()

        def fire_gathers(chunk, idx, dst, rows, sem):
            for j in range(RPW):
                m = chunk * RPW + j
                off = s * B_PER_W + (m // PERIOD) * 64
                pm = m % PERIOD
                for g in range(IDXW // LANES):
                    sl = pl.ds(g * LANES, LANES)
                    v = idx[j, sl]
                    u = jnp.bitwise_and(v, VB - 1)
                    idx[j, sl] = ((v - u)
                                  + jnp.left_shift(
                                      jnp.bitwise_and(u, OROWS - 1), 2)
                                  + jnp.right_shift(u, OROWS.bit_length() - 1))
                    dst[j, sl] = pat_v[pm, sl] + off
                pltpu.async_copy(table_hbm.at[idx.at[j]], rows.at[j], sem)

        def drain_gathers(idx, rows, sem):
            for j in range(RPW):
                pltpu.make_async_copy(table_hbm.at[idx.at[j]], rows.at[j],
                                      sem).wait()

        def fire_scatters(rows, dst, sem):
            for j in range(RPW):
                pltpu.async_copy(rows.at[j], acc.at[dst.at[j]], sem, add=True)

        def drain_scatters(rows, dst, sem):
            for j in range(RPW):
                pltpu.make_async_copy(rows.at[j], acc.at[dst.at[j]],
                                      sem).wait()

        pltpu.sync_copy(src_hbm.at[pl.ds(wid * ROWS_PER_W, RPW)], idx_a)
        fire_gathers(0, idx_a, dst_a, rows_a, sem_ga)

        def pair(i, _):
            fire_idx(2 * i + 1, idx_b, sem_ib)
            drain_gathers(idx_a, rows_a, sem_ga)
            fire_scatters(rows_a, dst_a, sem_sa)

            @pl.when(i > 0)
            def _():
                drain_scatters(rows_b, dst_b, sem_sb)

            wait_idx(idx_b, sem_ib)
            fire_gathers(2 * i + 1, idx_b, dst_b, rows_b, sem_gb)

            @pl.when(i < PAIRS - 1)
            def _():
                fire_idx(2 * i + 2, idx_a, sem_ia)

            drain_gathers(idx_b, rows_b, sem_gb)
            fire_scatters(rows_b, dst_b, sem_sb)
            drain_scatters(rows_a, dst_a, sem_sa)

            @pl.when(i < PAIRS - 1)
            def _():
                wait_idx(idx_a, sem_ia)
                fire_gathers(2 * i + 2, idx_a, dst_a, rows_a, sem_ga)

            return 0

        lax.fori_loop(0, PAIRS, pair, 0)
        drain_scatters(rows_b, dst_b, sem_sb)

        plsc.subcore_barrier()
        pltpu.sync_copy(acc.at[pl.ds(s * B_PER_W, B_PER_W)], stage)
        pltpu.sync_copy(stage, out_hbm.at[pl.ds(wid * B_PER_W, B_PER_W)])

    return k(emb_table, src_idx, pattern, zeros)


VB = 65536
NFULL = 1000000 // VB
TAIL = 1000000 - NFULL * VB
TGRID = NFULL + 1
OROWS = VB * D // IDXW
VEXT = TGRID * VB
NQ = VB // OROWS


def _detr_body(t_ref, tail_ref, o_ref):
    i = pl.program_id(0)
    r = lax.broadcasted_iota(jnp.int32, (IDXW, IDXW), 0)
    cc = lax.broadcasted_iota(jnp.int32, (IDXW, IDXW), 1)
    eye = (r == cc).astype(jnp.float32)
    dn = (((0,), (0,)), ((), ()))

    def emit(x):
        xs = jnp.concatenate(
            [x[:, q * OROWS:(q + 1) * OROWS] for q in range(NQ)], axis=0)
        o_ref[...] = jnp.transpose(xs)

    @pl.when(i < NFULL)
    def _():
        emit(t_ref[...])

    @pl.when(i == NFULL)
    def _():
        emit(jnp.concatenate(
            [tail_ref[...], jnp.zeros((D, VB - TAIL), jnp.float32)], axis=1))


def _tc_detranspose(t32):
    tail = t32[:, NFULL * VB:]
    return pl.pallas_call(
        _detr_body,
        grid=(TGRID,),
        in_specs=[
            pl.BlockSpec((D, VB), lambda i: (0, jnp.minimum(i, NFULL - 1))),
            pl.BlockSpec((D, TAIL), lambda i: (0, 0)),
        ],
        out_specs=pl.BlockSpec((OROWS, IDXW), lambda i: (i, 0)),
        out_shape=jax.ShapeDtypeStruct((VEXT * D // IDXW, IDXW), jnp.float32),
    )(t32, tail)


def _mlp_body(p_ref, w1_ref, b1_ref, w2_ref, b2_ref, o_ref):
    p = p_ref[...] * (1.0 / HL)
    h = jnp.dot(p, w1_ref[...], preferred_element_type=jnp.float32)
    h = jnp.maximum(h + b1_ref[...], 0.0)
    o = jnp.dot(h, w2_ref[...], preferred_element_type=jnp.float32)
    o_ref[...] = o + b2_ref[...]


def _tc_mlp(pooled, W1, b1, W2, b2):
    GB = 2048
    grid = (B // GB,)
    return pl.pallas_call(
        _mlp_body,
        grid=grid,
        in_specs=[
            pl.BlockSpec((GB, D), lambda i: (i, 0)),
            pl.BlockSpec((D, H), lambda i: (0, 0)),
            pl.BlockSpec((1, H), lambda i: (0, 0)),
            pl.BlockSpec((H, O), lambda i: (0, 0)),
            pl.BlockSpec((1, O), lambda i: (0, 0)),
        ],
        out_specs=pl.BlockSpec((GB, O), lambda i: (i, 0)),
        out_shape=jax.ShapeDtypeStruct((B, O), jnp.float32),
    )(pooled, W1, b1, W2, b2)


def kernel(text, emb_table, W1, b1, W2, b2):
    src_idx = text.astype(jnp.int32).reshape(IDX_ROWS, IDXW)
    pattern = (jnp.arange(PERIOD * IDXW, dtype=jnp.int32) // HL).reshape(
        PERIOD, IDXW)
    zeros = jnp.zeros((B_PER_W, D), jnp.float32)
    table_lin = _tc_detranspose(emb_table.T).reshape(VEXT, D)
    pooled = _sc_pool(table_lin, src_idx, pattern, zeros)
    return _tc_mlp(pooled, W1.astype(jnp.float32), b1.reshape(1, H),
                   W2.astype(jnp.float32), b2.reshape(1, O))
